# baseline (speedup 1.0000x reference)
import jax
import jax.numpy as jnp
from jax import lax
from jax.experimental import pallas as pl

N = 10000
E = 320000
F = 128
FH = 64
ED = 16
AVG_D_LOG = 3.4965
NPAD = 10240


def _ab_kernel(h_ref, ws_ref, wd_ref, b_ref, a0_ref, a1_ref, b0_ref, b1_ref):
    h = h_ref[...]
    a = jnp.dot(h, ws_ref[...], preferred_element_type=jnp.float32)
    b = jnp.dot(h, wd_ref[...], preferred_element_type=jnp.float32) + b_ref[...]
    a0_ref[...] = a[:, :FH]
    a1_ref[...] = a[:, FH:]
    b0_ref[...] = b[:, :FH]
    b1_ref[...] = b[:, FH:]


def _c_kernel(ea_ref, we_ref, c0_ref, c1_ref):
    c = jnp.dot(ea_ref[...], we_ref[...], preferred_element_type=jnp.float32)
    c0_ref[...] = c[:, :FH]
    c1_ref[...] = c[:, FH:]


def _post_kernel(h_ref, s0, s1, q0, q1, x0, x1, n0, n1, deg_ref,
                 wp_ref, bp_ref, o_ref, sum_ref, sq_ref):
    i = pl.program_id(0)
    deg = deg_ref[...]
    degs = jnp.maximum(deg, 1.0)
    has = deg > 0.0
    ssum = jnp.concatenate([s0[...], s1[...]], axis=1)
    sqsum = jnp.concatenate([q0[...], q1[...]], axis=1)
    mean = ssum / degs
    sq_mean = sqsum / degs
    mx = jnp.where(has, jnp.concatenate([x0[...], x1[...]], axis=1), 0.0)
    mn = jnp.where(has, jnp.concatenate([n0[...], n1[...]], axis=1), 0.0)
    std = jnp.sqrt(jax.nn.relu(sq_mean - mean * mean) + 1e-5)
    agg = jnp.concatenate([mean, mx, mn, std], axis=1)
    logd = jnp.log(degs + 1.0)
    wp = wp_ref[...]
    o = (jnp.dot(h_ref[...], wp[:F], preferred_element_type=jnp.float32)
         + jnp.dot(agg, wp[F:F + 512], preferred_element_type=jnp.float32)
         + (logd / AVG_D_LOG) * jnp.dot(agg, wp[F + 512:F + 1024],
                                        preferred_element_type=jnp.float32)
         + (AVG_D_LOG / logd) * jnp.dot(agg, wp[F + 1024:],
                                        preferred_element_type=jnp.float32))
    o = jax.nn.relu(o + bp_ref[...])
    o_ref[...] = o

    @pl.when(i == 0)
    def _():
        sum_ref[...] = jnp.zeros_like(sum_ref)
        sq_ref[...] = jnp.zeros_like(sq_ref)

    sum_ref[...] += jnp.sum(o, axis=0, keepdims=True)
    sq_ref[...] += jnp.sum(o * o, axis=0, keepdims=True)


def _bn_kernel(o_ref, sum_ref, sq_ref, g_ref, be_ref, out_ref):
    mu = sum_ref[...] / N
    var = sq_ref[...] / N - mu * mu
    inv = lax.rsqrt(var + 1e-5)
    out_ref[...] = (o_ref[...] - mu) * inv * g_ref[...] + be_ref[...]


def kernel(h, edge_index, edge_attr, W_pre, b_pre, W_post, b_post, gamma, beta):
    src = edge_index[0].astype(jnp.int32)
    dst = edge_index[1].astype(jnp.int32)
    Ws = W_pre[:F]
    Wd = W_pre[F:2 * F]
    We = W_pre[2 * F:]

    hp = jnp.concatenate([h, jnp.zeros((NPAD - N, F), jnp.float32)], axis=0)

    a0, a1, b0, b1 = pl.pallas_call(
        _ab_kernel,
        out_shape=(jax.ShapeDtypeStruct((NPAD, FH), jnp.float32),) * 4,
    )(hp, Ws, Wd, b_pre.reshape(1, F))

    EB = 3200
    c0, c1 = pl.pallas_call(
        _c_kernel,
        grid=(E // EB,),
        in_specs=[
            pl.BlockSpec((EB, ED), lambda i: (i, 0)),
            pl.BlockSpec((ED, F), lambda i: (0, 0)),
        ],
        out_specs=(
            pl.BlockSpec((EB, FH), lambda i: (i, 0)),
            pl.BlockSpec((EB, FH), lambda i: (i, 0)),
        ),
        out_shape=(
            jax.ShapeDtypeStruct((E, FH), jnp.float32),
            jax.ShapeDtypeStruct((E, FH), jnp.float32),
        ),
    )(edge_attr, We)

    a = jnp.concatenate([a0, a1], axis=1)
    b = jnp.concatenate([b0, b1], axis=1)
    c = jnp.concatenate([c0, c1], axis=1)
    e = jax.nn.relu(a[src] + b[dst] + c)

    ssum = jax.ops.segment_sum(e, dst, num_segments=N)
    sqsum = jax.ops.segment_sum(e * e, dst, num_segments=N)
    smax = jax.ops.segment_max(e, dst, num_segments=N)
    smin = jax.ops.segment_min(e, dst, num_segments=N)
    deg = jax.ops.segment_sum(jnp.ones((E,), jnp.float32), dst, num_segments=N)
    has = deg > 0
    smax = jnp.where(has[:, None], smax, 0.0)
    smin = jnp.where(has[:, None], smin, 0.0)

    R = 2000
    degc = deg.reshape(N, 1)
    o, colsum, colsq = pl.pallas_call(
        _post_kernel,
        grid=(N // R,),
        in_specs=[
            pl.BlockSpec((R, F), lambda i: (i, 0)),
            *[pl.BlockSpec((R, FH), lambda i: (i, 0)) for _ in range(8)],
            pl.BlockSpec((R, 1), lambda i: (i, 0)),
            pl.BlockSpec((13 * F, F), lambda i: (0, 0)),
            pl.BlockSpec((1, F), lambda i: (0, 0)),
        ],
        out_specs=(
            pl.BlockSpec((R, F), lambda i: (i, 0)),
            pl.BlockSpec((1, F), lambda i: (0, 0)),
            pl.BlockSpec((1, F), lambda i: (0, 0)),
        ),
        out_shape=(
            jax.ShapeDtypeStruct((N, F), jnp.float32),
            jax.ShapeDtypeStruct((1, F), jnp.float32),
            jax.ShapeDtypeStruct((1, F), jnp.float32),
        ),
    )(h, ssum[:, :FH], ssum[:, FH:], sqsum[:, :FH], sqsum[:, FH:],
      smax[:, :FH], smax[:, FH:], smin[:, :FH], smin[:, FH:],
      degc, W_post, b_post.reshape(1, F))

    return pl.pallas_call(
        _bn_kernel,
        out_shape=jax.ShapeDtypeStruct((N, F), jnp.float32),
    )(o, colsum, colsq, gamma.reshape(1, F), beta.reshape(1, F))
